# Initial kernel scaffold; baseline (speedup 1.0000x reference)
import jax
import jax.numpy as jnp
from jax.experimental import pallas as pl


def kernel(x, edge_index, Ws_self, Ws_neigh, bs, Wh_self, Wh_neigh, bh):
    raise NotImplementedError("write your pallas kernel here")


# speedup vs baseline: 5.3500x; 5.3500x over previous
import jax
import jax.numpy as jnp
from jax import lax
from jax.experimental import pallas as pl
from jax.experimental.pallas import tpu as pltpu
from jax.experimental.pallas import tpu_sc as plsc

N_NODES = 10000
NPAD = 10240
DIM = 128
N_EDGES = 320000
NUM_HEADS = 8

NC = 2
NS = 16
NW = NC * NS
CHUNK = 80
NSLICE = NW
SLICE_E = N_EDGES // NSLICE
NSUPER = 5
SUBCH = SLICE_E // (NSUPER * CHUNK)
ROWS_PER_TILE = NPAD // NS


def _sc_layer1(feat, src3, dst3, zeros_nd, ones_rows):
    def body(feat_hbm, src_hbm, dst_hbm, zeros_hbm, ones_hbm, out_hbm,
             src_v, dst_v, rows_v, acc_sh, sem):
        cid = lax.axis_index("c")
        sid = lax.axis_index("s")
        row0 = sid * ROWS_PER_TILE
        pltpu.sync_copy(zeros_hbm.at[pl.ds(row0, ROWS_PER_TILE)],
                        acc_sh.at[pl.ds(row0, ROWS_PER_TILE)])

        @pl.when(cid == 1)
        def _():
            pltpu.sync_copy(ones_hbm, rows_v)

        plsc.subcore_barrier()

        for t in range(2):
            def super_body(s, carry, t=t):
                w = sid * 2 + t
                pltpu.sync_copy(dst_hbm.at[w, s], dst_v)

                @pl.when(cid == 0)
                def _():
                    pltpu.sync_copy(src_hbm.at[w, s], src_v)

                def chunk_body(c, carry2):
                    @pl.when(cid == 0)
                    def _():
                        pltpu.async_copy(feat_hbm.at[src_v.at[c]], rows_v,
                                         sem).wait()
                    pltpu.sync_copy(rows_v, acc_sh.at[dst_v.at[c]], add=True)
                    return carry2

                lax.fori_loop(0, SUBCH, chunk_body, 0)
                return carry

            lax.fori_loop(0, NSUPER, super_body, 0)
        plsc.subcore_barrier()

        pltpu.sync_copy(acc_sh.at[pl.ds(row0, ROWS_PER_TILE)],
                        out_hbm.at[cid, pl.ds(row0, ROWS_PER_TILE)])

    mesh = plsc.VectorSubcoreMesh(core_axis_name="c", subcore_axis_name="s",
                                  num_cores=NC, num_subcores=NS)
    return pl.kernel(
        body,
        out_type=jax.ShapeDtypeStruct((NC, NPAD, DIM), jnp.float32),
        mesh=mesh,
        scratch_types=[
            pltpu.VMEM((SUBCH, CHUNK), jnp.int32),
            pltpu.VMEM((SUBCH, CHUNK), jnp.int32),
            pltpu.VMEM((CHUNK, DIM), jnp.float32),
            pltpu.VMEM_SHARED((NPAD, DIM), jnp.float32),
            pltpu.SemaphoreType.DMA,
        ],
        name="sc_sum_and_deg",
    )(feat, src3, dst3, zeros_nd, ones_rows)


def _sc_layer2(feat, src3, dst3, zeros_nd):
    def body(feat_hbm, src_hbm, dst_hbm, zeros_hbm, out_hbm,
             src_v, dst_v, rows_v, acc_sh, sem):
        cid = lax.axis_index("c")
        sid = lax.axis_index("s")
        wid = cid * NS + sid
        row0 = sid * ROWS_PER_TILE
        pltpu.sync_copy(zeros_hbm.at[pl.ds(row0, ROWS_PER_TILE)],
                        acc_sh.at[pl.ds(row0, ROWS_PER_TILE)])
        plsc.subcore_barrier()

        def super_body(s, carry):
            pltpu.sync_copy(src_hbm.at[wid, s], src_v)
            pltpu.sync_copy(dst_hbm.at[wid, s], dst_v)

            def chunk_body(c, carry2):
                pltpu.async_copy(feat_hbm.at[src_v.at[c]], rows_v, sem).wait()
                pltpu.sync_copy(rows_v, acc_sh.at[dst_v.at[c]], add=True)
                return carry2

            lax.fori_loop(0, SUBCH, chunk_body, 0)
            return carry

        lax.fori_loop(0, NSUPER, super_body, 0)
        plsc.subcore_barrier()

        pltpu.sync_copy(acc_sh.at[pl.ds(row0, ROWS_PER_TILE)],
                        out_hbm.at[cid, pl.ds(row0, ROWS_PER_TILE)])

    mesh = plsc.VectorSubcoreMesh(core_axis_name="c", subcore_axis_name="s",
                                  num_cores=NC, num_subcores=NS)
    return pl.kernel(
        body,
        out_type=jax.ShapeDtypeStruct((NC, NPAD, DIM), jnp.float32),
        mesh=mesh,
        scratch_types=[
            pltpu.VMEM((SUBCH, CHUNK), jnp.int32),
            pltpu.VMEM((SUBCH, CHUNK), jnp.int32),
            pltpu.VMEM((CHUNK, DIM), jnp.float32),
            pltpu.VMEM_SHARED((NPAD, DIM), jnp.float32),
            pltpu.SemaphoreType.DMA,
        ],
        name="sc_seg_sum",
    )(feat, src3, dst3, zeros_nd)


ROW_BLK = 1000
N_BLKS = N_NODES // ROW_BLK


def _tc_sage_body(x_ref, p_ref, deg_ref, wself_ref, wneigh_ref, b_ref, out_ref):
    deg = jnp.maximum(deg_ref[...], 1.0)
    psum = p_ref[0]
    for c in range(1, p_ref.shape[0]):
        psum = psum + p_ref[c]
    agg = psum / deg
    acc = jnp.dot(x_ref[...], wself_ref[...], preferred_element_type=jnp.float32)
    acc += jnp.dot(agg, wneigh_ref[...], preferred_element_type=jnp.float32)
    out_ref[...] = jnp.maximum(acc + b_ref[...], 0.0)


def _tc_sage(x, partials, deg, w_self, w_neigh, b):
    out_dim = w_self.shape[1]
    npart = partials.shape[0]
    return pl.pallas_call(
        _tc_sage_body,
        grid=(N_BLKS,),
        in_specs=[
            pl.BlockSpec((ROW_BLK, DIM), lambda i: (i, 0)),
            pl.BlockSpec((npart, ROW_BLK, DIM), lambda i: (0, i, 0)),
            pl.BlockSpec((ROW_BLK, DIM), lambda i: (i, 0)),
            pl.BlockSpec((DIM, out_dim), lambda i: (0, 0)),
            pl.BlockSpec((DIM, out_dim), lambda i: (0, 0)),
            pl.BlockSpec((1, out_dim), lambda i: (0, 0)),
        ],
        out_specs=pl.BlockSpec((ROW_BLK, out_dim), lambda i: (i, 0)),
        out_shape=jax.ShapeDtypeStruct((N_NODES, out_dim), jnp.float32),
    )(x, partials, deg, w_self, w_neigh, b)


def kernel(x, edge_index, Ws_self, Ws_neigh, bs, Wh_self, Wh_neigh, bh):
    src = edge_index[0].astype(jnp.int32).reshape(NSLICE, NSUPER, SUBCH, CHUNK)
    dst = edge_index[1].astype(jnp.int32).reshape(NSLICE, NSUPER, SUBCH, CHUNK)
    zeros_nd = jnp.zeros((NPAD, DIM), jnp.float32)
    ones_rows = jnp.ones((CHUNK, DIM), jnp.float32)

    out1 = _sc_layer1(x, src, dst, zeros_nd, ones_rows)
    p1 = out1[0:1]
    deg = out1[1]
    shared = _tc_sage(x, p1, deg, Ws_self, Ws_neigh, bs.reshape(1, DIM))

    p2 = _sc_layer2(shared, src, dst, zeros_nd)
    w_self_cat = jnp.transpose(Wh_self, (1, 0, 2)).reshape(DIM, NUM_HEADS * DIM)
    w_neigh_cat = jnp.transpose(Wh_neigh, (1, 0, 2)).reshape(DIM, NUM_HEADS * DIM)
    b_cat = bh.reshape(1, NUM_HEADS * DIM)
    heads = _tc_sage(shared, p2, deg, w_self_cat, w_neigh_cat, b_cat)
    return heads.reshape(N_NODES, NUM_HEADS, DIM)
